# baseline (speedup 1.0000x reference)
import functools
import math

import jax
import jax.numpy as jnp
from jax import lax
from jax.experimental import pallas as pl
from jax.experimental.pallas import tpu as pltpu
from jax.experimental.pallas import tpu_sc as plsc

_BATCH = 1024
_EMB = 128
_N = 100000
_S = 64.0
_M2 = 0.5
_EPS = 1e-7

_CB = 4096
_GRID = (_N + _CB - 1) // _CB

_COS_M = math.cos(_M2)
_SIN_M = math.sin(_M2)
_T_LO = math.cos(math.pi - _M2 - _EPS)
_COS_PI_EPS = math.cos(math.pi - _EPS)
_LOG_CLIP = math.log(1e-30)
_LN2 = math.log(2.0)
_C = _S / _LN2
_PAD_ROWS = float(_GRID * _CB - _N)


def _stream_kernel(emb_ref, w_ref, s_ref, ne_scr):
    b = pl.program_id(0)

    @pl.when(b == 0)
    def _init():
        e = emb_ref[...]
        nrm = jnp.sqrt(jnp.sum(e * e, axis=1, keepdims=True))
        ne_scr[...] = (_C * e) / jnp.maximum(nrm, 1e-12)
        s_ref[...] = jnp.zeros((_BATCH, _EMB), jnp.float32)

    w = w_ref[...]
    wn2 = jnp.sum(w * w, axis=1, keepdims=True)
    nw = w * jax.lax.rsqrt(jnp.maximum(wn2, 1e-24))
    rows = b * _CB + jax.lax.broadcasted_iota(jnp.int32, (_CB, 1), 0)
    nw = jnp.where(rows < _N, nw, 0.0)
    l2 = jax.lax.dot_general(
        ne_scr[...].astype(jnp.bfloat16), nw.astype(jnp.bfloat16),
        (((1,), (1,)), ((), ())),
        preferred_element_type=jnp.float32,
    )
    ev = jnp.exp2(l2)
    acc = ev[:, 0:_EMB]
    for k in range(1, _CB // _EMB):
        acc = acc + ev[:, k * _EMB:(k + 1) * _EMB]
    s_ref[...] += acc


def _epilogue_kernel(emb_ref, g_ref, s_ref, loss_ref):
    e = emb_ref[...]
    nrm = jnp.sqrt(jnp.sum(e * e, axis=1, keepdims=True))
    ne = (_C * e) / jnp.maximum(nrm, 1e-12)
    g = g_ref[...]
    gn2 = jnp.sum(g * g, axis=1, keepdims=True)
    ng = g * jax.lax.rsqrt(jnp.maximum(gn2, 1e-24))
    t2 = jnp.sum(ne * ng, axis=1, keepdims=True)
    s_tot = jnp.sum(s_ref[...], axis=1, keepdims=True) - _PAD_ROWS
    e_t = jnp.exp2(t2)
    t = jnp.clip(t2 * (1.0 / _C), -1.0 + _EPS, 1.0 - _EPS)
    cos_tm = t * _COS_M - jnp.sqrt(jnp.maximum(1.0 - t * t, 0.0)) * _SIN_M
    fc = jnp.where(t < _T_LO, _COS_PI_EPS, cos_tm)
    s_mod = s_tot - e_t + jnp.exp2(_C * fc)
    logp = _S * fc - jnp.log(s_mod)
    logp = jnp.maximum(logp, _LOG_CLIP)
    loss_ref[...] = -jnp.sum(logp, axis=(0, 1), keepdims=True) / _BATCH


def _make_sc_gather():
    info = plsc.get_sparse_core_info()
    nw_workers = info.num_cores * info.num_subcores
    b_per_w = _BATCH // nw_workers
    mesh = plsc.VectorSubcoreMesh(core_axis_name="c", subcore_axis_name="s")

    @functools.partial(
        pl.kernel, mesh=mesh,
        out_type=jax.ShapeDtypeStruct((_BATCH, _EMB), jnp.float32),
        scratch_types=[
            pltpu.VMEM((b_per_w,), jnp.int32),
            pltpu.VMEM((b_per_w, _EMB), jnp.float32),
            pltpu.SemaphoreType.DMA,
        ],
    )
    def sc_gather(w_hbm, idx_hbm, out_hbm, idx_v, rows_v, sem):
        wid = lax.axis_index("s") * info.num_cores + lax.axis_index("c")
        base = wid * b_per_w
        pltpu.sync_copy(idx_hbm.at[pl.ds(base, b_per_w)], idx_v)
        pltpu.async_copy(w_hbm.at[idx_v], rows_v, sem).wait()
        pltpu.sync_copy(rows_v, out_hbm.at[pl.ds(base, b_per_w)])

    return sc_gather


_sc_gather = _make_sc_gather()


@jax.jit
def kernel(local_embeddings, local_labels, weight):
    labels = local_labels.astype(jnp.int32)
    g = _sc_gather(weight, labels)
    s128 = pl.pallas_call(
        _stream_kernel,
        grid=(_GRID,),
        in_specs=[
            pl.BlockSpec((_BATCH, _EMB), lambda b: (0, 0)),
            pl.BlockSpec((_CB, _EMB), lambda b: (b, 0)),
        ],
        out_specs=pl.BlockSpec((_BATCH, _EMB), lambda b: (0, 0)),
        out_shape=jax.ShapeDtypeStruct((_BATCH, _EMB), jnp.float32),
        scratch_shapes=[
            pltpu.VMEM((_BATCH, _EMB), jnp.float32),
        ],
        compiler_params=pltpu.CompilerParams(
            dimension_semantics=("arbitrary",),
        ),
    )(local_embeddings, weight)
    loss = pl.pallas_call(
        _epilogue_kernel,
        out_shape=jax.ShapeDtypeStruct((1, 1), jnp.float32),
    )(local_embeddings, g, s128)
    return loss[0, 0]


# speedup vs baseline: 1.1075x; 1.1075x over previous
import functools
import math

import jax
import jax.numpy as jnp
from jax import lax
from jax.experimental import pallas as pl
from jax.experimental.pallas import tpu as pltpu
from jax.experimental.pallas import tpu_sc as plsc

_BATCH = 1024
_EMB = 128
_N = 100000
_S = 64.0
_M2 = 0.5
_EPS = 1e-7

_CB = 4096
_GRID = (_N + _CB - 1) // _CB

_COS_M = math.cos(_M2)
_SIN_M = math.sin(_M2)
_T_LO = math.cos(math.pi - _M2 - _EPS)
_COS_PI_EPS = math.cos(math.pi - _EPS)
_LOG_CLIP = math.log(1e-30)
_LN2 = math.log(2.0)
_C = _S / _LN2
_PAD_ROWS = float(_GRID * _CB - _N)


def _stream_kernel(emb_ref, w_ref, s_ref, ne_scr):
    b = pl.program_id(0)

    @pl.when(b == 0)
    def _init():
        e = emb_ref[...]
        nrm = jnp.sqrt(jnp.sum(e * e, axis=1, keepdims=True))
        ne_scr[...] = (_C * e) / jnp.maximum(nrm, 1e-12)
        s_ref[...] = jnp.zeros((_BATCH, _EMB), jnp.float32)

    rows = b * _CB + jax.lax.broadcasted_iota(jnp.int32, (_CB, 1), 0)
    w = jnp.where(rows < _N, w_ref[...], 0.0)
    ones8 = jnp.ones((8, _EMB), jnp.float32)
    wn2r = jax.lax.dot_general(
        ones8, w * w,
        (((1,), (1,)), ((), ())),
        preferred_element_type=jnp.float32,
    )
    inv = jax.lax.rsqrt(jnp.maximum(wn2r[0:1, :], 1e-24))
    l2 = jax.lax.dot_general(
        ne_scr[...], w,
        (((1,), (1,)), ((), ())),
        preferred_element_type=jnp.float32,
    )
    ev = jnp.exp2(l2 * inv)
    acc = ev[:, 0:_EMB]
    for k in range(1, _CB // _EMB):
        acc = acc + ev[:, k * _EMB:(k + 1) * _EMB]
    s_ref[...] += acc


def _epilogue_kernel(emb_ref, g_ref, s_ref, loss_ref):
    e = emb_ref[...]
    nrm = jnp.sqrt(jnp.sum(e * e, axis=1, keepdims=True))
    ne = (_C * e) / jnp.maximum(nrm, 1e-12)
    g = g_ref[...]
    gn2 = jnp.sum(g * g, axis=1, keepdims=True)
    ng = g * jax.lax.rsqrt(jnp.maximum(gn2, 1e-24))
    t2 = jnp.sum(ne * ng, axis=1, keepdims=True)
    s_tot = jnp.sum(s_ref[...], axis=1, keepdims=True) - _PAD_ROWS
    e_t = jnp.exp2(t2)
    t = jnp.clip(t2 * (1.0 / _C), -1.0 + _EPS, 1.0 - _EPS)
    cos_tm = t * _COS_M - jnp.sqrt(jnp.maximum(1.0 - t * t, 0.0)) * _SIN_M
    fc = jnp.where(t < _T_LO, _COS_PI_EPS, cos_tm)
    s_mod = s_tot - e_t + jnp.exp2(_C * fc)
    logp = _S * fc - jnp.log(s_mod)
    logp = jnp.maximum(logp, _LOG_CLIP)
    loss_ref[...] = -jnp.sum(logp, axis=(0, 1), keepdims=True) / _BATCH


def _make_sc_gather():
    info = plsc.get_sparse_core_info()
    nw_workers = info.num_cores * info.num_subcores
    b_per_w = _BATCH // nw_workers
    mesh = plsc.VectorSubcoreMesh(core_axis_name="c", subcore_axis_name="s")

    @functools.partial(
        pl.kernel, mesh=mesh,
        out_type=jax.ShapeDtypeStruct((_BATCH, _EMB), jnp.float32),
        scratch_types=[
            pltpu.VMEM((b_per_w,), jnp.int32),
            pltpu.VMEM((b_per_w, _EMB), jnp.float32),
            pltpu.SemaphoreType.DMA,
        ],
    )
    def sc_gather(w_hbm, idx_hbm, out_hbm, idx_v, rows_v, sem):
        wid = lax.axis_index("s") * info.num_cores + lax.axis_index("c")
        base = wid * b_per_w
        pltpu.sync_copy(idx_hbm.at[pl.ds(base, b_per_w)], idx_v)
        pltpu.async_copy(w_hbm.at[idx_v], rows_v, sem).wait()
        pltpu.sync_copy(rows_v, out_hbm.at[pl.ds(base, b_per_w)])

    return sc_gather


_sc_gather = _make_sc_gather()


@jax.jit
def kernel(local_embeddings, local_labels, weight):
    labels = local_labels.astype(jnp.int32)
    g = _sc_gather(weight, labels)
    s128 = pl.pallas_call(
        _stream_kernel,
        grid=(_GRID,),
        in_specs=[
            pl.BlockSpec((_BATCH, _EMB), lambda b: (0, 0)),
            pl.BlockSpec((_CB, _EMB), lambda b: (b, 0)),
        ],
        out_specs=pl.BlockSpec((_BATCH, _EMB), lambda b: (0, 0)),
        out_shape=jax.ShapeDtypeStruct((_BATCH, _EMB), jnp.float32),
        scratch_shapes=[
            pltpu.VMEM((_BATCH, _EMB), jnp.float32),
        ],
        compiler_params=pltpu.CompilerParams(
            dimension_semantics=("arbitrary",),
        ),
    )(local_embeddings, weight)
    loss = pl.pallas_call(
        _epilogue_kernel,
        out_shape=jax.ShapeDtypeStruct((1, 1), jnp.float32),
    )(local_embeddings, g, s128)
    return loss[0, 0]


# speedup vs baseline: 1.1138x; 1.0057x over previous
import functools
import math

import jax
import jax.numpy as jnp
from jax import lax
from jax.experimental import pallas as pl
from jax.experimental.pallas import tpu as pltpu
from jax.experimental.pallas import tpu_sc as plsc

_BATCH = 1024
_EMB = 128
_N = 100000
_S = 64.0
_M2 = 0.5
_EPS = 1e-7

_CB = 5120
_GRID = (_N + _CB - 1) // _CB

_COS_M = math.cos(_M2)
_SIN_M = math.sin(_M2)
_T_LO = math.cos(math.pi - _M2 - _EPS)
_COS_PI_EPS = math.cos(math.pi - _EPS)
_LOG_CLIP = math.log(1e-30)
_LN2 = math.log(2.0)
_C = _S / _LN2
_PAD_ROWS = float(_GRID * _CB - _N)


def _stream_kernel(emb_ref, w_ref, s_ref, ne_scr):
    b = pl.program_id(0)

    @pl.when(b == 0)
    def _init():
        e = emb_ref[...]
        nrm = jnp.sqrt(jnp.sum(e * e, axis=1, keepdims=True))
        ne_scr[...] = (_C * e) / jnp.maximum(nrm, 1e-12)
        s_ref[...] = jnp.zeros((_BATCH, _EMB), jnp.float32)

    rows = b * _CB + jax.lax.broadcasted_iota(jnp.int32, (_CB, 1), 0)
    w = jnp.where(rows < _N, w_ref[...], 0.0)
    ones8 = jnp.ones((8, _EMB), jnp.float32)
    wn2r = jax.lax.dot_general(
        ones8, w * w,
        (((1,), (1,)), ((), ())),
        preferred_element_type=jnp.float32,
    )
    inv = jax.lax.rsqrt(jnp.maximum(wn2r[0:1, :], 1e-24))
    l2 = jax.lax.dot_general(
        ne_scr[...], w,
        (((1,), (1,)), ((), ())),
        preferred_element_type=jnp.float32,
    )
    ev = jnp.exp2(l2 * inv)
    acc = ev[:, 0:_EMB]
    for k in range(1, _CB // _EMB):
        acc = acc + ev[:, k * _EMB:(k + 1) * _EMB]
    s_ref[...] += acc


def _epilogue_kernel(emb_ref, g_ref, s_ref, loss_ref):
    e = emb_ref[...]
    nrm = jnp.sqrt(jnp.sum(e * e, axis=1, keepdims=True))
    ne = (_C * e) / jnp.maximum(nrm, 1e-12)
    g = g_ref[...]
    gn2 = jnp.sum(g * g, axis=1, keepdims=True)
    ng = g * jax.lax.rsqrt(jnp.maximum(gn2, 1e-24))
    t2 = jnp.sum(ne * ng, axis=1, keepdims=True)
    s_tot = jnp.sum(s_ref[...], axis=1, keepdims=True) - _PAD_ROWS
    e_t = jnp.exp2(t2)
    t = jnp.clip(t2 * (1.0 / _C), -1.0 + _EPS, 1.0 - _EPS)
    cos_tm = t * _COS_M - jnp.sqrt(jnp.maximum(1.0 - t * t, 0.0)) * _SIN_M
    fc = jnp.where(t < _T_LO, _COS_PI_EPS, cos_tm)
    s_mod = s_tot - e_t + jnp.exp2(_C * fc)
    logp = _S * fc - jnp.log(s_mod)
    logp = jnp.maximum(logp, _LOG_CLIP)
    loss_ref[...] = -jnp.sum(logp, axis=(0, 1), keepdims=True) / _BATCH


def _make_sc_gather():
    info = plsc.get_sparse_core_info()
    nw_workers = info.num_cores * info.num_subcores
    b_per_w = _BATCH // nw_workers
    mesh = plsc.VectorSubcoreMesh(core_axis_name="c", subcore_axis_name="s")

    @functools.partial(
        pl.kernel, mesh=mesh,
        out_type=jax.ShapeDtypeStruct((_BATCH, _EMB), jnp.float32),
        scratch_types=[
            pltpu.VMEM((b_per_w,), jnp.int32),
            pltpu.VMEM((b_per_w, _EMB), jnp.float32),
            pltpu.SemaphoreType.DMA,
        ],
    )
    def sc_gather(w_hbm, idx_hbm, out_hbm, idx_v, rows_v, sem):
        wid = lax.axis_index("s") * info.num_cores + lax.axis_index("c")
        base = wid * b_per_w
        pltpu.sync_copy(idx_hbm.at[pl.ds(base, b_per_w)], idx_v)
        pltpu.async_copy(w_hbm.at[idx_v], rows_v, sem).wait()
        pltpu.sync_copy(rows_v, out_hbm.at[pl.ds(base, b_per_w)])

    return sc_gather


_sc_gather = _make_sc_gather()


@jax.jit
def kernel(local_embeddings, local_labels, weight):
    labels = local_labels.astype(jnp.int32)
    g = _sc_gather(weight, labels)
    s128 = pl.pallas_call(
        _stream_kernel,
        grid=(_GRID,),
        in_specs=[
            pl.BlockSpec((_BATCH, _EMB), lambda b: (0, 0)),
            pl.BlockSpec((_CB, _EMB), lambda b: (b, 0)),
        ],
        out_specs=pl.BlockSpec((_BATCH, _EMB), lambda b: (0, 0)),
        out_shape=jax.ShapeDtypeStruct((_BATCH, _EMB), jnp.float32),
        scratch_shapes=[
            pltpu.VMEM((_BATCH, _EMB), jnp.float32),
        ],
        compiler_params=pltpu.CompilerParams(
            dimension_semantics=("arbitrary",),
        ),
    )(local_embeddings, weight)
    loss = pl.pallas_call(
        _epilogue_kernel,
        out_shape=jax.ShapeDtypeStruct((1, 1), jnp.float32),
    )(local_embeddings, g, s128)
    return loss[0, 0]


# speedup vs baseline: 1.1337x; 1.0179x over previous
import functools
import math

import jax
import jax.numpy as jnp
from jax import lax
from jax.experimental import pallas as pl
from jax.experimental.pallas import tpu as pltpu
from jax.experimental.pallas import tpu_sc as plsc

_BATCH = 1024
_EMB = 128
_N = 100000
_S = 64.0
_M2 = 0.5
_EPS = 1e-7

_CB = 6272
_GRID = (_N + _CB - 1) // _CB

_COS_M = math.cos(_M2)
_SIN_M = math.sin(_M2)
_T_LO = math.cos(math.pi - _M2 - _EPS)
_COS_PI_EPS = math.cos(math.pi - _EPS)
_LOG_CLIP = math.log(1e-30)
_LN2 = math.log(2.0)
_C = _S / _LN2
_PAD_ROWS = float(_GRID * _CB - _N)


def _stream_kernel(emb_ref, w_ref, s_ref, ne_scr):
    b = pl.program_id(0)

    @pl.when(b == 0)
    def _init():
        e = emb_ref[...]
        nrm = jnp.sqrt(jnp.sum(e * e, axis=1, keepdims=True))
        ne_scr[...] = (_C * e) / jnp.maximum(nrm, 1e-12)
        s_ref[...] = jnp.zeros((_BATCH, _EMB), jnp.float32)

    rows = b * _CB + jax.lax.broadcasted_iota(jnp.int32, (_CB, 1), 0)
    w = jnp.where(rows < _N, w_ref[...], 0.0)
    ones8 = jnp.ones((8, _EMB), jnp.float32)
    wn2r = jax.lax.dot_general(
        ones8, w * w,
        (((1,), (1,)), ((), ())),
        preferred_element_type=jnp.float32,
    )
    inv = jax.lax.rsqrt(jnp.maximum(wn2r[0:1, :], 1e-24))
    l2 = jax.lax.dot_general(
        ne_scr[...], w,
        (((1,), (1,)), ((), ())),
        preferred_element_type=jnp.float32,
    )
    ev = jnp.exp2(l2 * inv)
    acc = ev[:, 0:_EMB]
    for k in range(1, _CB // _EMB):
        acc = acc + ev[:, k * _EMB:(k + 1) * _EMB]
    s_ref[...] += acc


def _epilogue_kernel(emb_ref, g_ref, s_ref, loss_ref):
    e = emb_ref[...]
    nrm = jnp.sqrt(jnp.sum(e * e, axis=1, keepdims=True))
    ne = (_C * e) / jnp.maximum(nrm, 1e-12)
    g = g_ref[...]
    gn2 = jnp.sum(g * g, axis=1, keepdims=True)
    ng = g * jax.lax.rsqrt(jnp.maximum(gn2, 1e-24))
    t2 = jnp.sum(ne * ng, axis=1, keepdims=True)
    s_tot = jnp.sum(s_ref[...], axis=1, keepdims=True) - _PAD_ROWS
    e_t = jnp.exp2(t2)
    t = jnp.clip(t2 * (1.0 / _C), -1.0 + _EPS, 1.0 - _EPS)
    cos_tm = t * _COS_M - jnp.sqrt(jnp.maximum(1.0 - t * t, 0.0)) * _SIN_M
    fc = jnp.where(t < _T_LO, _COS_PI_EPS, cos_tm)
    s_mod = s_tot - e_t + jnp.exp2(_C * fc)
    logp = _S * fc - jnp.log(s_mod)
    logp = jnp.maximum(logp, _LOG_CLIP)
    loss_ref[...] = -jnp.sum(logp, axis=(0, 1), keepdims=True) / _BATCH


def _make_sc_gather():
    info = plsc.get_sparse_core_info()
    nw_workers = info.num_cores * info.num_subcores
    b_per_w = _BATCH // nw_workers
    mesh = plsc.VectorSubcoreMesh(core_axis_name="c", subcore_axis_name="s")

    @functools.partial(
        pl.kernel, mesh=mesh,
        out_type=jax.ShapeDtypeStruct((_BATCH, _EMB), jnp.float32),
        scratch_types=[
            pltpu.VMEM((b_per_w,), jnp.int32),
            pltpu.VMEM((b_per_w, _EMB), jnp.float32),
            pltpu.SemaphoreType.DMA,
        ],
    )
    def sc_gather(w_hbm, idx_hbm, out_hbm, idx_v, rows_v, sem):
        wid = lax.axis_index("s") * info.num_cores + lax.axis_index("c")
        base = wid * b_per_w
        pltpu.sync_copy(idx_hbm.at[pl.ds(base, b_per_w)], idx_v)
        pltpu.async_copy(w_hbm.at[idx_v], rows_v, sem).wait()
        pltpu.sync_copy(rows_v, out_hbm.at[pl.ds(base, b_per_w)])

    return sc_gather


_sc_gather = _make_sc_gather()


@jax.jit
def kernel(local_embeddings, local_labels, weight):
    labels = local_labels.astype(jnp.int32)
    g = _sc_gather(weight, labels)
    s128 = pl.pallas_call(
        _stream_kernel,
        grid=(_GRID,),
        in_specs=[
            pl.BlockSpec((_BATCH, _EMB), lambda b: (0, 0)),
            pl.BlockSpec((_CB, _EMB), lambda b: (b, 0)),
        ],
        out_specs=pl.BlockSpec((_BATCH, _EMB), lambda b: (0, 0)),
        out_shape=jax.ShapeDtypeStruct((_BATCH, _EMB), jnp.float32),
        scratch_shapes=[
            pltpu.VMEM((_BATCH, _EMB), jnp.float32),
        ],
        compiler_params=pltpu.CompilerParams(
            dimension_semantics=("arbitrary",),
        ),
    )(local_embeddings, weight)
    loss = pl.pallas_call(
        _epilogue_kernel,
        out_shape=jax.ShapeDtypeStruct((1, 1), jnp.float32),
    )(local_embeddings, g, s128)
    return loss[0, 0]


# speedup vs baseline: 1.1432x; 1.0084x over previous
import functools
import math

import jax
import jax.numpy as jnp
from jax import lax
from jax.experimental import pallas as pl
from jax.experimental.pallas import tpu as pltpu
from jax.experimental.pallas import tpu_sc as plsc

_BATCH = 1024
_EMB = 128
_N = 100000
_S = 64.0
_M2 = 0.5
_EPS = 1e-7

_CB = 7168
_GRID = (_N + _CB - 1) // _CB

_COS_M = math.cos(_M2)
_SIN_M = math.sin(_M2)
_T_LO = math.cos(math.pi - _M2 - _EPS)
_COS_PI_EPS = math.cos(math.pi - _EPS)
_LOG_CLIP = math.log(1e-30)
_LN2 = math.log(2.0)
_C = _S / _LN2
_PAD_ROWS = float(_GRID * _CB - _N)


def _stream_kernel(emb_ref, w_ref, s_ref, ne_scr):
    b = pl.program_id(0)

    @pl.when(b == 0)
    def _init():
        e = emb_ref[...]
        nrm = jnp.sqrt(jnp.sum(e * e, axis=1, keepdims=True))
        ne_scr[...] = (_C * e) / jnp.maximum(nrm, 1e-12)
        s_ref[...] = jnp.zeros((_BATCH, _EMB), jnp.float32)

    rows = b * _CB + jax.lax.broadcasted_iota(jnp.int32, (_CB, 1), 0)
    w = jnp.where(rows < _N, w_ref[...], 0.0)
    ones8 = jnp.ones((8, _EMB), jnp.float32)
    wn2r = jax.lax.dot_general(
        ones8, w * w,
        (((1,), (1,)), ((), ())),
        preferred_element_type=jnp.float32,
    )
    inv = jax.lax.rsqrt(jnp.maximum(wn2r[0:1, :], 1e-24))
    l2 = jax.lax.dot_general(
        ne_scr[...], w,
        (((1,), (1,)), ((), ())),
        preferred_element_type=jnp.float32,
    )
    ev = jnp.exp2(l2 * inv)
    acc = ev[:, 0:_EMB]
    for k in range(1, _CB // _EMB):
        acc = acc + ev[:, k * _EMB:(k + 1) * _EMB]
    s_ref[...] += acc


def _epilogue_kernel(emb_ref, g_ref, s_ref, loss_ref):
    e = emb_ref[...]
    nrm = jnp.sqrt(jnp.sum(e * e, axis=1, keepdims=True))
    ne = (_C * e) / jnp.maximum(nrm, 1e-12)
    g = g_ref[...]
    gn2 = jnp.sum(g * g, axis=1, keepdims=True)
    ng = g * jax.lax.rsqrt(jnp.maximum(gn2, 1e-24))
    t2 = jnp.sum(ne * ng, axis=1, keepdims=True)
    s_tot = jnp.sum(s_ref[...], axis=1, keepdims=True) - _PAD_ROWS
    e_t = jnp.exp2(t2)
    t = jnp.clip(t2 * (1.0 / _C), -1.0 + _EPS, 1.0 - _EPS)
    cos_tm = t * _COS_M - jnp.sqrt(jnp.maximum(1.0 - t * t, 0.0)) * _SIN_M
    fc = jnp.where(t < _T_LO, _COS_PI_EPS, cos_tm)
    s_mod = s_tot - e_t + jnp.exp2(_C * fc)
    logp = _S * fc - jnp.log(s_mod)
    logp = jnp.maximum(logp, _LOG_CLIP)
    loss_ref[...] = -jnp.sum(logp, axis=(0, 1), keepdims=True) / _BATCH


def _make_sc_gather():
    info = plsc.get_sparse_core_info()
    nw_workers = info.num_cores * info.num_subcores
    b_per_w = _BATCH // nw_workers
    mesh = plsc.VectorSubcoreMesh(core_axis_name="c", subcore_axis_name="s")

    @functools.partial(
        pl.kernel, mesh=mesh,
        out_type=jax.ShapeDtypeStruct((_BATCH, _EMB), jnp.float32),
        scratch_types=[
            pltpu.VMEM((b_per_w,), jnp.int32),
            pltpu.VMEM((b_per_w, _EMB), jnp.float32),
            pltpu.SemaphoreType.DMA,
        ],
    )
    def sc_gather(w_hbm, idx_hbm, out_hbm, idx_v, rows_v, sem):
        wid = lax.axis_index("s") * info.num_cores + lax.axis_index("c")
        base = wid * b_per_w
        pltpu.sync_copy(idx_hbm.at[pl.ds(base, b_per_w)], idx_v)
        pltpu.async_copy(w_hbm.at[idx_v], rows_v, sem).wait()
        pltpu.sync_copy(rows_v, out_hbm.at[pl.ds(base, b_per_w)])

    return sc_gather


_sc_gather = _make_sc_gather()


@jax.jit
def kernel(local_embeddings, local_labels, weight):
    labels = local_labels.astype(jnp.int32)
    g = _sc_gather(weight, labels)
    s128 = pl.pallas_call(
        _stream_kernel,
        grid=(_GRID,),
        in_specs=[
            pl.BlockSpec((_BATCH, _EMB), lambda b: (0, 0)),
            pl.BlockSpec((_CB, _EMB), lambda b: (b, 0)),
        ],
        out_specs=pl.BlockSpec((_BATCH, _EMB), lambda b: (0, 0)),
        out_shape=jax.ShapeDtypeStruct((_BATCH, _EMB), jnp.float32),
        scratch_shapes=[
            pltpu.VMEM((_BATCH, _EMB), jnp.float32),
        ],
        compiler_params=pltpu.CompilerParams(
            dimension_semantics=("arbitrary",),
        ),
    )(local_embeddings, weight)
    loss = pl.pallas_call(
        _epilogue_kernel,
        out_shape=jax.ShapeDtypeStruct((1, 1), jnp.float32),
    )(local_embeddings, g, s128)
    return loss[0, 0]
